# baseline (speedup 1.0000x reference)
import functools

import jax
import jax.numpy as jnp
from jax import lax
from jax.experimental import pallas as pl
from jax.experimental.pallas import tpu as pltpu
from jax.experimental.pallas import tpu_sc as plsc


def _mu_kernel(B, V, D, NC, L, bpw):
    qtr = bpw // 4
    mesh = plsc.VectorSubcoreMesh(core_axis_name="c", subcore_axis_name="s")

    @functools.partial(
        pl.kernel,
        mesh=mesh,
        compiler_params=pltpu.CompilerParams(use_tc_tiling_on_sc=True),
        out_type=jax.ShapeDtypeStruct((B, D), jnp.float32),
        scratch_types=[
            pltpu.VMEM((bpw,), jnp.int32),
            pltpu.VMEM((2, qtr, 8, D), jnp.float32),
            pltpu.VMEM((bpw, D), jnp.float32),
            pltpu.SemaphoreType.DMA,
            pltpu.SemaphoreType.DMA,
        ],
    )
    def run(idx_hbm, tbl_hbm, out_hbm, idx_v, slab_v, row_v, sem0, sem1):
        wid = lax.axis_index("s") * NC + lax.axis_index("c")
        base = wid * bpw
        pltpu.sync_copy(idx_hbm.at[pl.ds(base, bpw)], idx_v)
        sems = (sem0, sem1)

        def issue(p):
            cps = []
            for i in range(qtr // L):
                rv = idx_v[pl.ds(p * qtr + i * L, L)]
                for l in range(L):
                    cps.append(pltpu.async_copy(
                        tbl_hbm.at[rv[l] >> 3],
                        slab_v.at[p % 2, i * L + l], sems[p % 2]))
            return cps

        pend = {0: issue(0)}
        for p in range(4):
            if p + 1 < 4:
                pend[p + 1] = issue(p + 1)
            for cp in pend.pop(p):
                cp.wait()
            for i in range(qtr // L):
                rv = idx_v[pl.ds(p * qtr + i * L, L)]
                for l in range(L):
                    jj = i * L + l
                    j = p * qtr + jj
                    q = rv[l] & 7
                    for cb in range(D // L):
                        row_v[j, pl.ds(cb * L, L)] = (
                            slab_v[p % 2, jj, q, pl.ds(cb * L, L)])

        pltpu.sync_copy(row_v, out_hbm.at[pl.ds(base, bpw)])

    return run


def _sig_concat_kernel(B, V, D, NC, L, bpw):
    qtr = bpw // 4
    mesh = plsc.VectorSubcoreMesh(core_axis_name="c", subcore_axis_name="s")

    @functools.partial(
        pl.kernel,
        mesh=mesh,
        compiler_params=pltpu.CompilerParams(use_tc_tiling_on_sc=True),
        out_type=jax.ShapeDtypeStruct((B, 2 * D), jnp.float32),
        scratch_types=[
            pltpu.VMEM((bpw,), jnp.int32),
            pltpu.VMEM((2, qtr, 8, D), jnp.float32),
            pltpu.VMEM((bpw, D), jnp.float32),
            pltpu.VMEM((bpw, 2 * D), jnp.float32),
            pltpu.SemaphoreType.DMA,
            pltpu.SemaphoreType.DMA,
            pltpu.SemaphoreType.DMA,
        ],
    )
    def run(idx_hbm, tbl_hbm, mu_hbm, out_hbm,
            idx_v, slab_v, mu_v, row_v, sem0, sem1, sem_mu):
        wid = lax.axis_index("s") * NC + lax.axis_index("c")
        base = wid * bpw
        pltpu.sync_copy(idx_hbm.at[pl.ds(base, bpw)], idx_v)
        mu_cp = pltpu.async_copy(mu_hbm.at[pl.ds(base, bpw)], mu_v, sem_mu)
        sems = (sem0, sem1)

        def issue(p):
            cps = []
            for i in range(qtr // L):
                rv = idx_v[pl.ds(p * qtr + i * L, L)]
                for l in range(L):
                    cps.append(pltpu.async_copy(
                        tbl_hbm.at[rv[l] >> 3],
                        slab_v.at[p % 2, i * L + l], sems[p % 2]))
            return cps

        pend = {0: issue(0)}
        mu_cp.wait()
        for j in range(bpw):
            for cb in range(D // L):
                row_v[j, pl.ds(cb * L, L)] = mu_v[j, pl.ds(cb * L, L)]

        for p in range(4):
            if p + 1 < 4:
                pend[p + 1] = issue(p + 1)
            for cp in pend.pop(p):
                cp.wait()
            for i in range(qtr // L):
                rv = idx_v[pl.ds(p * qtr + i * L, L)]
                for l in range(L):
                    jj = i * L + l
                    j = p * qtr + jj
                    q = rv[l] & 7
                    for cb in range(D // L):
                        sv = slab_v[p % 2, jj, q, pl.ds(cb * L, L)]
                        row_v[j, pl.ds(D + cb * L, L)] = (
                            jnp.maximum(sv, 0.0)
                            + jnp.exp(jnp.minimum(sv, 0.0)))

        pltpu.sync_copy(row_v, out_hbm.at[pl.ds(base, bpw)])

    return run


def kernel(idx, mu_weight, sigma_weight):
    B = idx.shape[0]
    V, D = mu_weight.shape
    info = plsc.get_sparse_core_info()
    NC, NS, L = info.num_cores, info.num_subcores, info.num_lanes
    NW = NC * NS
    assert B % (L * NW) == 0 and D % L == 0 and V % 8 == 0
    bpw = B // NW

    mu_run = _mu_kernel(B, V, D, NC, L, bpw)
    sig_run = _sig_concat_kernel(B, V, D, NC, L, bpw)
    mu_emb = mu_run(idx, mu_weight.reshape(V // 8, 8, D))
    return sig_run(idx, sigma_weight.reshape(V // 8, 8, D), mu_emb)


# speedup vs baseline: 1.0736x; 1.0736x over previous
import functools

import jax
import jax.numpy as jnp
from jax import lax
from jax.experimental import pallas as pl
from jax.experimental.pallas import tpu as pltpu
from jax.experimental.pallas import tpu_sc as plsc


def _mu_kernel(B, V, D, NC, L, bpw):
    qtr = bpw // 4
    mesh = plsc.VectorSubcoreMesh(core_axis_name="c", subcore_axis_name="s")

    @functools.partial(
        pl.kernel,
        mesh=mesh,
        compiler_params=pltpu.CompilerParams(use_tc_tiling_on_sc=True),
        out_type=jax.ShapeDtypeStruct((B, D), jnp.float32),
        scratch_types=[
            pltpu.VMEM((bpw,), jnp.int32),
            pltpu.VMEM((2, qtr, 8, D), jnp.float32),
            pltpu.VMEM((bpw, D), jnp.float32),
            pltpu.SemaphoreType.DMA,
            pltpu.SemaphoreType.DMA,
        ],
    )
    def run(idx_hbm, tbl_hbm, out_hbm, idx_v, slab_v, row_v, sem0, sem1):
        wid = lax.axis_index("s") * NC + lax.axis_index("c")
        base = wid * bpw
        pltpu.sync_copy(idx_hbm.at[pl.ds(base, bpw)], idx_v)
        sems = (sem0, sem1)

        def issue(p):
            cps = []
            for i in range(qtr // L):
                rv = idx_v[pl.ds(p * qtr + i * L, L)]
                for l in range(L):
                    cps.append(pltpu.async_copy(
                        tbl_hbm.at[rv[l] >> 3],
                        slab_v.at[p % 2, i * L + l], sems[p % 2]))
            return cps

        pend = {0: issue(0)}
        for p in range(4):
            if p + 1 < 4:
                pend[p + 1] = issue(p + 1)
            for cp in pend.pop(p):
                cp.wait()
            for i in range(qtr // L):
                rv = idx_v[pl.ds(p * qtr + i * L, L)]
                for l in range(L):
                    jj = i * L + l
                    j = p * qtr + jj
                    q = rv[l] & 7
                    for cb in range(D // L):
                        row_v[j, pl.ds(cb * L, L)] = (
                            slab_v[p % 2, jj, q, pl.ds(cb * L, L)])

        pltpu.sync_copy(row_v, out_hbm.at[pl.ds(base, bpw)])

    return run


def _sig_concat_kernel(B, V, D, NC, L, bpw):
    qtr = bpw // 4
    mesh = plsc.VectorSubcoreMesh(core_axis_name="c", subcore_axis_name="s")

    @functools.partial(
        pl.kernel,
        mesh=mesh,
        compiler_params=pltpu.CompilerParams(use_tc_tiling_on_sc=True),
        out_type=jax.ShapeDtypeStruct((B, 2 * D), jnp.float32),
        scratch_types=[
            pltpu.VMEM((bpw,), jnp.int32),
            pltpu.VMEM((2, qtr, 8, D), jnp.float32),
            pltpu.VMEM((bpw, D), jnp.float32),
            pltpu.VMEM((bpw, 2 * D), jnp.float32),
            pltpu.SemaphoreType.DMA,
            pltpu.SemaphoreType.DMA,
            pltpu.SemaphoreType.DMA,
        ],
    )
    def run(idx_hbm, tbl_hbm, mu_hbm, out_hbm,
            idx_v, slab_v, mu_v, row_v, sem0, sem1, sem_mu):
        wid = lax.axis_index("s") * NC + lax.axis_index("c")
        base = wid * bpw
        pltpu.sync_copy(idx_hbm.at[pl.ds(base, bpw)], idx_v)
        mu_cp = pltpu.async_copy(mu_hbm.at[pl.ds(base, bpw)], mu_v, sem_mu)
        sems = (sem0, sem1)

        def issue(p):
            cps = []
            for i in range(qtr // L):
                rv = idx_v[pl.ds(p * qtr + i * L, L)]
                for l in range(L):
                    t8 = pl.multiple_of((rv[l] >> 3) * 8, 8)
                    cps.append(pltpu.async_copy(
                        tbl_hbm.at[pl.ds(t8, 8), :],
                        slab_v.at[p % 2, i * L + l], sems[p % 2]))
            return cps

        pend = {0: issue(0)}
        mu_cp.wait()
        for j in range(bpw):
            for cb in range(D // L):
                row_v[j, pl.ds(cb * L, L)] = mu_v[j, pl.ds(cb * L, L)]

        for p in range(4):
            if p + 1 < 4:
                pend[p + 1] = issue(p + 1)
            for cp in pend.pop(p):
                cp.wait()
            for i in range(qtr // L):
                rv = idx_v[pl.ds(p * qtr + i * L, L)]
                for l in range(L):
                    jj = i * L + l
                    j = p * qtr + jj
                    q = rv[l] & 7
                    for cb in range(D // L):
                        sv = slab_v[p % 2, jj, q, pl.ds(cb * L, L)]
                        row_v[j, pl.ds(D + cb * L, L)] = (
                            jnp.maximum(sv, 0.0)
                            + jnp.exp(jnp.minimum(sv, 0.0)))

        pltpu.sync_copy(row_v, out_hbm.at[pl.ds(base, bpw)])

    return run


def kernel(idx, mu_weight, sigma_weight):
    B = idx.shape[0]
    V, D = mu_weight.shape
    info = plsc.get_sparse_core_info()
    NC, NS, L = info.num_cores, info.num_subcores, info.num_lanes
    NW = NC * NS
    assert B % (L * NW) == 0 and D % L == 0 and V % 8 == 0
    bpw = B // NW

    mu_run = _mu_kernel(B, V, D, NC, L, bpw)
    sig_run = _sig_concat_kernel(B, V, D, NC, L, bpw)
    mu_emb = mu_run(idx, mu_weight.reshape(V // 8, 8, D))
    return sig_run(idx, sigma_weight, mu_emb)
